# baseline (speedup 1.0000x reference)
import jax
import jax.numpy as jnp
from jax.experimental import pallas as pl
from jax.experimental.pallas import tpu as pltpu

_ROWS_PER_STEP = 2048
_STREAMS = 2


def _gather_kernel(*refs, rows, streams):
    idx_refs = refs[:streams]
    tab_ref = refs[streams]
    out_ref = refs[streams + 1]
    for k in range(rows // streams):
        for j in range(streams):
            v = idx_refs[j][0, 0, k]
            out_ref[k * streams + j, :] = tab_ref[v, 0, :]


def kernel(table, x):
    V, D = table.shape
    B, F = x.shape
    N = B * F

    R = min(_ROWS_PER_STEP, N)
    n_steps = pl.cdiv(N, R)
    N_pad = n_steps * R

    flat_idx = x.reshape(-1).astype(jnp.int32)
    if N_pad != N:
        flat_idx = jnp.pad(flat_idx, (0, N_pad - N))

    J = _STREAMS if R % _STREAMS == 0 else 1
    K = R // J
    Kp = K + 1
    idx_kj = flat_idx.reshape(n_steps, K, J)
    idx_streams = [
        jnp.pad(idx_kj[:, :, j], ((0, 0), (0, Kp - K))).reshape(n_steps, 1, Kp)
        for j in range(J)
    ]

    tab3 = table.reshape(V, 1, D)

    out = pl.pallas_call(
        lambda *refs: _gather_kernel(*refs, rows=R, streams=J),
        out_shape=jax.ShapeDtypeStruct((N_pad, D), table.dtype),
        grid=(n_steps,),
        in_specs=[
            pl.BlockSpec((1, 1, Kp), lambda i: (i, 0, 0),
                         memory_space=pltpu.SMEM)
            for _ in range(J)
        ] + [
            pl.BlockSpec((V, 1, D), lambda i: (0, 0, 0)),
        ],
        out_specs=pl.BlockSpec((R, D), lambda i: (i, 0)),
        compiler_params=pltpu.CompilerParams(
            dimension_semantics=("arbitrary",),
            vmem_limit_bytes=48 * 1024 * 1024),
        cost_estimate=pl.CostEstimate(
            flops=0,
            transcendentals=0,
            bytes_accessed=N_pad * 4 + V * D * 4 + N_pad * D * 4),
    )(*idx_streams, tab3)

    if N_pad != N:
        out = out[:N]
    return out.reshape(B, F, D)


# speedup vs baseline: 1.4589x; 1.4589x over previous
import jax
import jax.numpy as jnp
from jax.experimental import pallas as pl
from jax.experimental.pallas import tpu as pltpu

_ROWS_PER_STEP = 4096
_STREAMS = 1


def _gather_kernel(*refs, rows, streams):
    idx_refs = refs[:streams]
    tab_ref = refs[streams]
    out_ref = refs[streams + 1]
    for k in range(rows // streams):
        for j in range(streams):
            v = idx_refs[j][0, 0, k]
            out_ref[k * streams + j, :] = tab_ref[v, 0, :]


def kernel(table, x):
    V, D = table.shape
    B, F = x.shape
    N = B * F

    R = min(_ROWS_PER_STEP, N)
    n_steps = pl.cdiv(N, R)
    N_pad = n_steps * R

    flat_idx = x.reshape(-1).astype(jnp.int32)
    if N_pad != N:
        flat_idx = jnp.pad(flat_idx, (0, N_pad - N))

    J = _STREAMS if R % _STREAMS == 0 else 1
    K = R // J
    Kp = K + 1 if J > 1 else K
    idx_kj = flat_idx.reshape(n_steps, K, J)
    idx_streams = [
        jnp.pad(idx_kj[:, :, j], ((0, 0), (0, Kp - K))).reshape(n_steps, 1, Kp)
        for j in range(J)
    ]

    tab3 = table.reshape(V, 1, D)

    out = pl.pallas_call(
        lambda *refs: _gather_kernel(*refs, rows=R, streams=J),
        out_shape=jax.ShapeDtypeStruct((N_pad, D), table.dtype),
        grid=(n_steps,),
        in_specs=[
            pl.BlockSpec((1, 1, Kp), lambda i: (i, 0, 0),
                         memory_space=pltpu.SMEM)
            for _ in range(J)
        ] + [
            pl.BlockSpec((V, 1, D), lambda i: (0, 0, 0)),
        ],
        out_specs=pl.BlockSpec((R, D), lambda i: (i, 0)),
        compiler_params=pltpu.CompilerParams(
            dimension_semantics=("arbitrary",),
            vmem_limit_bytes=48 * 1024 * 1024),
        cost_estimate=pl.CostEstimate(
            flops=0,
            transcendentals=0,
            bytes_accessed=N_pad * 4 + V * D * 4 + N_pad * D * 4),
    )(*idx_streams, tab3)

    if N_pad != N:
        out = out[:N]
    return out.reshape(B, F, D)


# speedup vs baseline: 2.0428x; 1.4002x over previous
import jax
import jax.numpy as jnp
from jax import lax
from jax.experimental import pallas as pl
from jax.experimental.pallas import tpu as pltpu

_ROWS_PER_STEP = 2048
_STREAMS = 4


def _make_body(n_steps, J, K, V, D):
    def body(idx_hbm, tab_ref, out_ref, *scr):
        smem = scr[:2 * J]
        sem = scr[2 * J]
        i = pl.program_id(0)
        p = lax.rem(i, 2)
        pn = 1 - p

        def copy(step, par, j):
            return pltpu.make_async_copy(
                idx_hbm.at[pl.ds(step, 1), pl.ds(j, 1), :],
                smem[par * J + j],
                sem.at[par, j])

        @pl.when(i == 0)
        def _():
            for j in range(J):
                copy(0, 0, j).start()

        for par in range(2):
            @pl.when(jnp.logical_and(i + 1 < n_steps, pn == par))
            def _(par=par):
                for j in range(J):
                    copy(i + 1, par, j).start()

        for par in range(2):
            @pl.when(p == par)
            def _(par=par):
                for j in range(J):
                    copy(i, par, j).wait()
                for k in range(K):
                    for j in range(J):
                        v = smem[par * J + j][0, 0, k]
                        out_ref[j * K + k, :] = tab_ref[v, 0, :]

    return body


def kernel(table, x):
    V, D = table.shape
    B, F = x.shape
    N = B * F

    R = min(_ROWS_PER_STEP, N)
    n_steps = pl.cdiv(N, R)
    N_pad = n_steps * R

    flat_idx = x.reshape(-1).astype(jnp.int32)
    if N_pad != N:
        flat_idx = jnp.pad(flat_idx, (0, N_pad - N))

    J = _STREAMS if R % _STREAMS == 0 else 1
    K = R // J
    idx_v = flat_idx.reshape(n_steps, J, K)

    tab3 = table.reshape(V, 1, D)

    out = pl.pallas_call(
        _make_body(n_steps, J, K, V, D),
        out_shape=jax.ShapeDtypeStruct((N_pad, D), table.dtype),
        grid=(n_steps,),
        in_specs=[
            pl.BlockSpec(memory_space=pl.ANY),
            pl.BlockSpec((V, 1, D), lambda i: (0, 0, 0)),
        ],
        out_specs=pl.BlockSpec((R, D), lambda i: (i, 0)),
        scratch_shapes=(
            [pltpu.SMEM((1, 1, K), jnp.int32) for _ in range(2 * J)]
            + [pltpu.SemaphoreType.DMA((2, J))]
        ),
        compiler_params=pltpu.CompilerParams(
            dimension_semantics=("arbitrary",),
            vmem_limit_bytes=48 * 1024 * 1024),
        cost_estimate=pl.CostEstimate(
            flops=0,
            transcendentals=0,
            bytes_accessed=N_pad * 4 + V * D * 4 + N_pad * D * 4),
    )(idx_v, tab3)

    if N_pad != N:
        out = out[:N]
    return out.reshape(B, F, D)
